# baseline (speedup 1.0000x reference)
import functools

import jax
import jax.numpy as jnp
from jax import lax
from jax.experimental import pallas as pl
from jax.experimental.pallas import tpu as pltpu
from jax.experimental.pallas import tpu_sc as plsc

N_ENTITY = 10000
N_REL = 12
DIM = 128
N_MOVIE = 6924
N_EDGE = 320000
B = 256
S = 32

NC = 2
NS = 16
CH = 128
SEG = N_ENTITY * N_REL
SEG_PAD = 16 * 7552
CHUNKS_ALL = N_EDGE // CH
CHUNKS_SC = CHUNKS_ALL // NC
F32 = jnp.float32
I32 = jnp.int32


def _rgcn_partials(src, dst, rel, wflat, seeds, root, bias):
    mesh = plsc.VectorSubcoreMesh(core_axis_name="c", subcore_axis_name="s")
    CB = 1024
    NB = 320
    SUP = CHUNKS_SC
    NQ = 80
    EPC = CHUNKS_SC * CH

    @functools.partial(
        pl.kernel,
        out_type=(
            jax.ShapeDtypeStruct((N_ENTITY, DIM), F32),
            jax.ShapeDtypeStruct((N_ENTITY, DIM), F32),
            jax.ShapeDtypeStruct((B * S, DIM), F32),
            jax.ShapeDtypeStruct((B * S, DIM), F32),
        ),
        mesh=mesh,
        scratch_types=[
            pltpu.VMEM_SHARED((SEG_PAD,), F32),
            pltpu.VMEM_SHARED((N_ENTITY, DIM), F32),
            pltpu.VMEM((CB,), I32),
            pltpu.VMEM((CB,), I32),
            pltpu.VMEM((8, CH), I32),
            pltpu.VMEM((2, CH), I32),
            pltpu.VMEM((2, CH), I32),
            pltpu.VMEM((2, CH), I32),
            pltpu.VMEM((2, CH), I32),
            pltpu.VMEM((2, CH), I32),
            pltpu.VMEM((2, CH), I32),
            pltpu.VMEM((2, CH), F32),
            pltpu.VMEM((144,), F32),
            pltpu.VMEM((144,), F32),
            pltpu.VMEM((CH,), F32),
            pltpu.VMEM((1888,), F32),
            pltpu.VMEM((2 * CH, DIM), F32),
            pltpu.SemaphoreType.DMA,
            pltpu.SemaphoreType.DMA,
            pltpu.SemaphoreType.DMA,
            pltpu.SemaphoreType.DMA,
            pltpu.SemaphoreType.DMA,
            pltpu.SemaphoreType.DMA,
            pltpu.SemaphoreType.DMA,
        ],
    )
    def k1(src_ref, dst_ref, rel_ref, wflat_ref, seed_ref, root_ref,
           bias_ref, pa_ref, pb_ref, ha_ref, hb_ref,
           cnt_sh, agg_sh, cdst, crel, cseg, esrc, edst, erel,
           gidx, segidx, dstidx, cval, scale0, scale1, ones, cstage, rows,
           sem_e0, sem_e1, sem_g0, sem_g1, sem_s0, sem_s1, sem_c):
        c = lax.axis_index("c")
        s = lax.axis_index("s")
        sem_e = (sem_e0, sem_e1)
        sem_g = (sem_g0, sem_g1)
        sem_s = (sem_s0, sem_s1)

        for k in range(CH // 16):
            ones[pl.ds(k * 16, 16)] = jnp.ones((16,), F32)

        def _zrow(i, carry):
            for k in range(DIM // 16):
                rows[i, pl.ds(k * 16, 16)] = jnp.zeros((16,), F32)
            return carry

        lax.fori_loop(0, 80, _zrow, 0)

        def _zc(i, carry):
            cstage[pl.ds(i * 16, 16)] = jnp.zeros((16,), F32)
            return carry

        lax.fori_loop(0, 1888 // 16, _zc, 0)

        pltpu.sync_copy(bias_ref, scale0.at[pl.ds(0, DIM)])
        for j in range(8):
            cid = s + NS * j

            @pl.when(cid < N_ENTITY // 80)
            def _():
                @pl.when(c == 0)
                def _():
                    pltpu.sync_copy(root_ref.at[pl.ds(cid * 80, 80)],
                                    rows.at[pl.ds(CH, 80)])

                    def _radd(i, carry):
                        for k in range(DIM // 16):
                            col = pl.ds(k * 16, 16)
                            rows[CH + i, col] = (rows[CH + i, col]
                                                 + scale0[col])
                        return carry

                    lax.fori_loop(0, 80, _radd, 0, unroll=8)
                    pltpu.sync_copy(rows.at[pl.ds(CH, 80)],
                                    agg_sh.at[pl.ds(cid * 80, 80)])

                @pl.when(c == 1)
                def _():
                    pltpu.sync_copy(rows.at[pl.ds(0, 80)],
                                    agg_sh.at[pl.ds(cid * 80, 80)])
        for j in range(4):
            pltpu.sync_copy(cstage, cnt_sh.at[pl.ds(s * 7552 + j * 1888,
                                                    1888)])
        plsc.subcore_barrier()

        def _cblk(jb, carry):
            b = s + NS * jb
            orig = b * CB
            bbase = jnp.minimum(orig, N_EDGE - CB)
            e1 = pltpu.async_copy(dst_ref.at[pl.ds(bbase, CB)], cdst,
                                  sem_e[0])
            e2 = pltpu.async_copy(rel_ref.at[pl.ds(bbase, CB)], crel,
                                  sem_e[1])
            e1.wait()
            e2.wait()
            prev = None
            for k in range(CB // CH):
                for g in range(8):
                    off = k * CH + g * 16
                    d16 = cdst[pl.ds(off, 16)]
                    r16 = crel[pl.ds(off, 16)]
                    seg = d16 * N_REL + r16
                    ci = ((bbase + off) >= orig).astype(I32)
                    cseg[k, pl.ds(g * 16, 16)] = seg * ci + (1 - ci) * SEG
                if prev is not None:
                    prev.wait()
                prev = pltpu.async_copy(ones, cnt_sh.at[cseg.at[k]], sem_c,
                                        add=True)
            prev.wait()
            return carry

        lax.fori_loop(0, NB // NS, _cblk, 0)
        plsc.subcore_barrier()

        for j in range(4):
            pltpu.sync_copy(cnt_sh.at[pl.ds(s * 7552 + j * 1888, 1888)],
                            cstage)

            def _inv(i, carry):
                v = cstage[pl.ds(i * 16, 16)]
                cstage[pl.ds(i * 16, 16)] = 1.0 / jnp.maximum(v, 1.0)
                return carry

            lax.fori_loop(0, 1888 // 16, _inv, 0, unroll=4)
            pltpu.sync_copy(cstage, cnt_sh.at[pl.ds(s * 7552 + j * 1888,
                                                    1888)])
        plsc.subcore_barrier()

        def _pair(q2, carry):
            eds = []
            for sl in (0, 1):
                su = s + NS * (2 * q2 + sl)
                base = c * EPC + jnp.minimum(su, SUP - 1) * CH
                eds.append((
                    pltpu.async_copy(src_ref.at[pl.ds(base, CH)],
                                     esrc.at[sl], sem_e[sl]),
                    pltpu.async_copy(dst_ref.at[pl.ds(base, CH)],
                                     edst.at[sl], sem_e[sl]),
                    pltpu.async_copy(rel_ref.at[pl.ds(base, CH)],
                                     erel.at[sl], sem_e[sl]),
                ))
            gds = []
            for sl in (0, 1):
                for d in eds[sl]:
                    d.wait()
                for g in range(8):
                    off = g * 16
                    s16 = esrc[sl, pl.ds(off, 16)]
                    d16 = edst[sl, pl.ds(off, 16)]
                    r16 = erel[sl, pl.ds(off, 16)]
                    gidx[sl, pl.ds(off, 16)] = r16 * N_ENTITY + s16
                    segidx[sl, pl.ds(off, 16)] = d16 * N_REL + r16
                    dstidx[sl, pl.ds(off, 16)] = d16
                pltpu.async_copy(cnt_sh.at[segidx.at[sl]], cval.at[sl],
                                 sem_c).wait()
                gds.append(pltpu.async_copy(wflat_ref.at[gidx.at[sl]],
                                            rows.at[pl.ds(sl * CH, CH)],
                                            sem_g[sl]))
            prev_scatter = None
            for sl in (0, 1):
                gds[sl].wait()
                su = s + NS * (2 * q2 + sl)
                vf = (su < SUP).astype(F32)
                scale = scale0 if sl == 0 else scale1
                for g in range(8):
                    v16 = cval[sl, pl.ds(g * 16, 16)]
                    scale[pl.ds(g * 16, 16)] = v16 * vf

                def _scl(e, carry2):
                    sv = scale[pl.ds(e, 16)][0]
                    for g in range(8):
                        rows[sl * CH + e, pl.ds(g * 16, 16)] = (
                            rows[sl * CH + e, pl.ds(g * 16, 16)] * sv)
                    return carry2

                lax.fori_loop(0, CH, _scl, 0, unroll=8)
                if prev_scatter is not None:
                    prev_scatter.wait()
                prev_scatter = pltpu.async_copy(
                    rows.at[pl.ds(sl * CH, CH)], agg_sh.at[dstidx.at[sl]],
                    sem_s[sl], add=True)
            prev_scatter.wait()
            return carry

        lax.fori_loop(0, NQ // 2, _pair, 0)
        plsc.subcore_barrier()

        for j in range(8):
            cid = s + NS * j

            @pl.when(cid < N_ENTITY // 80)
            def _():
                @pl.when(c == 0)
                def _():
                    pltpu.sync_copy(agg_sh.at[pl.ds(cid * 80, 80)],
                                    pa_ref.at[pl.ds(cid * 80, 80)])

                @pl.when(c == 1)
                def _():
                    pltpu.sync_copy(agg_sh.at[pl.ds(cid * 80, 80)],
                                    pb_ref.at[pl.ds(cid * 80, 80)])

        for t in range(B * S // (NS * CH)):
            sbase = s * (B * S // NS) + t * CH
            pltpu.sync_copy(seed_ref.at[pl.ds(sbase, CH)], esrc.at[0])
            pltpu.async_copy(agg_sh.at[esrc.at[0]],
                             rows.at[pl.ds(0, CH)], sem_g[0]).wait()

            @pl.when(c == 0)
            def _():
                pltpu.sync_copy(rows.at[pl.ds(0, CH)],
                                ha_ref.at[pl.ds(sbase, CH)])

            @pl.when(c == 1)
            def _():
                pltpu.sync_copy(rows.at[pl.ds(0, CH)],
                                hb_ref.at[pl.ds(sbase, CH)])

    return k1(src, dst, rel, wflat, seeds, root, bias)


def _head(pa, pb, ha, hb, attn_a, attn_b, labels2d, out_bias2d):
    def body(p0_ref, p1_ref, ha_ref, hb_ref, a_ref,
             b_ref, lab_ref, ob_ref, scores_ref, loss_ref):
        mf = p0_ref[...][:N_MOVIE] + p1_ref[...][:N_MOVIE]
        hm = ha_ref[...] + hb_ref[...]
        t = jnp.tanh(jnp.dot(hm, a_ref[...], preferred_element_type=F32))
        e = jnp.dot(t, b_ref[...], preferred_element_type=F32)
        r_ids = lax.broadcasted_iota(I32, (B, B * S), 0)
        c_ids = lax.broadcasted_iota(I32, (B, B * S), 1)
        g = jnp.where(c_ids // S == r_ids, 1.0, 0.0).astype(F32)
        w = jnp.exp(e)
        denom = jnp.dot(g, w, preferred_element_type=F32)
        u = jnp.dot(g, w * hm, preferred_element_type=F32) / denom
        scores = lax.dot_general(u, mf, (((1,), (1,)), ((), ())),
                                 preferred_element_type=F32) + ob_ref[...]
        scores_ref[...] = scores
        m = jnp.max(scores, axis=1, keepdims=True)
        lse = jnp.log(jnp.sum(jnp.exp(scores - m), axis=1, keepdims=True)) + m
        logp = scores - lse
        col = lax.broadcasted_iota(I32, (B, N_MOVIE), 1)
        picked = jnp.sum(jnp.where(col == lab_ref[...], logp, 0.0),
                         axis=1, keepdims=True)
        loss = -jnp.sum(picked) / B
        loss_ref[...] = jnp.broadcast_to(loss, (1, 1))

    return pl.pallas_call(
        body,
        out_shape=(
            jax.ShapeDtypeStruct((B, N_MOVIE), F32),
            jax.ShapeDtypeStruct((1, 1), F32),
        ),
    )(pa, pb, ha, hb, attn_a, attn_b, labels2d, out_bias2d)


def kernel(seed_sets, labels, edge_index, edge_type, weight, root,
           rgcn_bias, attn_a, attn_b, out_bias):
    src = edge_index[0]
    dst = edge_index[1]
    wflat = weight.reshape(N_REL * N_ENTITY, DIM)
    pa, pb, ha, hb = _rgcn_partials(src, dst, edge_type, wflat,
                                    seed_sets.reshape(-1), root, rgcn_bias)
    scores, loss = _head(pa, pb, ha, hb, attn_a, attn_b,
                         labels.reshape(B, 1), out_bias.reshape(1, N_MOVIE))
    base_loss = loss.reshape(())
    return scores, base_loss, base_loss


# speedup vs baseline: 1.0403x; 1.0403x over previous
import functools

import jax
import jax.numpy as jnp
from jax import lax
from jax.experimental import pallas as pl
from jax.experimental.pallas import tpu as pltpu
from jax.experimental.pallas import tpu_sc as plsc

N_ENTITY = 10000
N_REL = 12
DIM = 128
N_MOVIE = 6924
N_EDGE = 320000
B = 256
S = 32

NC = 2
NS = 16
CH = 128
SEG = N_ENTITY * N_REL
SEG_PAD = 16 * 7552
CHUNKS_ALL = N_EDGE // CH
CHUNKS_SC = CHUNKS_ALL // NC
F32 = jnp.float32
I32 = jnp.int32


def _rgcn_partials(src, dst, rel, wflat):
    mesh = plsc.VectorSubcoreMesh(core_axis_name="c", subcore_axis_name="s")
    CB = 1024
    NB = 320
    SUP = CHUNKS_SC
    NQ = 80
    EPC = CHUNKS_SC * CH

    @functools.partial(
        pl.kernel,
        out_type=(
            jax.ShapeDtypeStruct((N_ENTITY, DIM), F32),
            jax.ShapeDtypeStruct((N_ENTITY, DIM), F32),
        ),
        mesh=mesh,
        scratch_types=[
            pltpu.VMEM_SHARED((SEG_PAD,), F32),
            pltpu.VMEM_SHARED((N_ENTITY, DIM), F32),
            pltpu.VMEM((CB,), I32),
            pltpu.VMEM((CB,), I32),
            pltpu.VMEM((8, CH), I32),
            pltpu.VMEM((2, CH), I32),
            pltpu.VMEM((2, CH), I32),
            pltpu.VMEM((2, CH), I32),
            pltpu.VMEM((2, CH), I32),
            pltpu.VMEM((2, CH), I32),
            pltpu.VMEM((2, CH), I32),
            pltpu.VMEM((2, CH), F32),
            pltpu.VMEM((144,), F32),
            pltpu.VMEM((144,), F32),
            pltpu.VMEM((CH,), F32),
            pltpu.VMEM((1888,), F32),
            pltpu.VMEM((2 * CH, DIM), F32),
            pltpu.SemaphoreType.DMA,
            pltpu.SemaphoreType.DMA,
            pltpu.SemaphoreType.DMA,
            pltpu.SemaphoreType.DMA,
            pltpu.SemaphoreType.DMA,
            pltpu.SemaphoreType.DMA,
            pltpu.SemaphoreType.DMA,
        ],
    )
    def k1(src_ref, dst_ref, rel_ref, wflat_ref, pa_ref, pb_ref,
           cnt_sh, agg_sh, cdst, crel, cseg, esrc, edst, erel,
           gidx, segidx, dstidx, cval, scale0, scale1, ones, cstage, rows,
           sem_e0, sem_e1, sem_g0, sem_g1, sem_s0, sem_s1, sem_c):
        c = lax.axis_index("c")
        s = lax.axis_index("s")
        sem_e = (sem_e0, sem_e1)
        sem_g = (sem_g0, sem_g1)
        sem_s = (sem_s0, sem_s1)

        for k in range(CH // 16):
            ones[pl.ds(k * 16, 16)] = jnp.ones((16,), F32)

        def _zrow(i, carry):
            for k in range(DIM // 16):
                rows[i, pl.ds(k * 16, 16)] = jnp.zeros((16,), F32)
            return carry

        lax.fori_loop(0, 80, _zrow, 0)

        def _zc(i, carry):
            cstage[pl.ds(i * 16, 16)] = jnp.zeros((16,), F32)
            return carry

        lax.fori_loop(0, 1888 // 16, _zc, 0)

        for j in range(8):
            cid = s + NS * j

            @pl.when(cid < N_ENTITY // 80)
            def _():
                pltpu.sync_copy(rows.at[pl.ds(0, 80)],
                                agg_sh.at[pl.ds(cid * 80, 80)])
        for j in range(4):
            pltpu.sync_copy(cstage, cnt_sh.at[pl.ds(s * 7552 + j * 1888,
                                                    1888)])
        plsc.subcore_barrier()

        def _cblk(jb, carry):
            b = s + NS * jb
            orig = b * CB
            bbase = jnp.minimum(orig, N_EDGE - CB)
            e1 = pltpu.async_copy(dst_ref.at[pl.ds(bbase, CB)], cdst,
                                  sem_e[0])
            e2 = pltpu.async_copy(rel_ref.at[pl.ds(bbase, CB)], crel,
                                  sem_e[1])
            e1.wait()
            e2.wait()
            prev = None
            for k in range(CB // CH):
                for g in range(8):
                    off = k * CH + g * 16
                    d16 = cdst[pl.ds(off, 16)]
                    r16 = crel[pl.ds(off, 16)]
                    seg = d16 * N_REL + r16
                    ci = ((bbase + off) >= orig).astype(I32)
                    cseg[k, pl.ds(g * 16, 16)] = seg * ci + (1 - ci) * SEG
                if prev is not None:
                    prev.wait()
                prev = pltpu.async_copy(ones, cnt_sh.at[cseg.at[k]], sem_c,
                                        add=True)
            prev.wait()
            return carry

        lax.fori_loop(0, NB // NS, _cblk, 0)
        plsc.subcore_barrier()

        for j in range(4):
            pltpu.sync_copy(cnt_sh.at[pl.ds(s * 7552 + j * 1888, 1888)],
                            cstage)

            def _inv(i, carry):
                v = cstage[pl.ds(i * 16, 16)]
                cstage[pl.ds(i * 16, 16)] = 1.0 / jnp.maximum(v, 1.0)
                return carry

            lax.fori_loop(0, 1888 // 16, _inv, 0, unroll=4)
            pltpu.sync_copy(cstage, cnt_sh.at[pl.ds(s * 7552 + j * 1888,
                                                    1888)])
        plsc.subcore_barrier()

        def _pair(q2, carry):
            eds = []
            for sl in (0, 1):
                su = s + NS * (2 * q2 + sl)
                base = c * EPC + jnp.minimum(su, SUP - 1) * CH
                eds.append((
                    pltpu.async_copy(src_ref.at[pl.ds(base, CH)],
                                     esrc.at[sl], sem_e[sl]),
                    pltpu.async_copy(dst_ref.at[pl.ds(base, CH)],
                                     edst.at[sl], sem_e[sl]),
                    pltpu.async_copy(rel_ref.at[pl.ds(base, CH)],
                                     erel.at[sl], sem_e[sl]),
                ))
            gds = []
            for sl in (0, 1):
                for d in eds[sl]:
                    d.wait()
                for g in range(8):
                    off = g * 16
                    s16 = esrc[sl, pl.ds(off, 16)]
                    d16 = edst[sl, pl.ds(off, 16)]
                    r16 = erel[sl, pl.ds(off, 16)]
                    gidx[sl, pl.ds(off, 16)] = r16 * N_ENTITY + s16
                    segidx[sl, pl.ds(off, 16)] = d16 * N_REL + r16
                    dstidx[sl, pl.ds(off, 16)] = d16
                pltpu.async_copy(cnt_sh.at[segidx.at[sl]], cval.at[sl],
                                 sem_c).wait()
                gds.append(pltpu.async_copy(wflat_ref.at[gidx.at[sl]],
                                            rows.at[pl.ds(sl * CH, CH)],
                                            sem_g[sl]))
            prev_scatter = None
            for sl in (0, 1):
                gds[sl].wait()
                su = s + NS * (2 * q2 + sl)
                vf = (su < SUP).astype(F32)
                scale = scale0 if sl == 0 else scale1
                for g in range(8):
                    v16 = cval[sl, pl.ds(g * 16, 16)]
                    scale[pl.ds(g * 16, 16)] = v16 * vf

                def _scl(e, carry2):
                    sv = scale[pl.ds(e, 16)][0]
                    for g in range(8):
                        rows[sl * CH + e, pl.ds(g * 16, 16)] = (
                            rows[sl * CH + e, pl.ds(g * 16, 16)] * sv)
                    return carry2

                lax.fori_loop(0, CH, _scl, 0, unroll=8)
                if prev_scatter is not None:
                    prev_scatter.wait()
                prev_scatter = pltpu.async_copy(
                    rows.at[pl.ds(sl * CH, CH)], agg_sh.at[dstidx.at[sl]],
                    sem_s[sl], add=True)
            prev_scatter.wait()
            return carry

        lax.fori_loop(0, NQ // 2, _pair, 0)
        plsc.subcore_barrier()

        for j in range(8):
            cid = s + NS * j

            @pl.when(cid < N_ENTITY // 80)
            def _():
                @pl.when(c == 0)
                def _():
                    pltpu.sync_copy(agg_sh.at[pl.ds(cid * 80, 80)],
                                    pa_ref.at[pl.ds(cid * 80, 80)])

                @pl.when(c == 1)
                def _():
                    pltpu.sync_copy(agg_sh.at[pl.ds(cid * 80, 80)],
                                    pb_ref.at[pl.ds(cid * 80, 80)])

    return k1(src, dst, rel, wflat)


def _seed_gather(seeds, pa, pb, root, bias):
    mesh = plsc.VectorSubcoreMesh(core_axis_name="c", subcore_axis_name="s")
    n_seed = B * S
    per_w = n_seed // (NC * NS)

    @functools.partial(
        pl.kernel,
        out_type=jax.ShapeDtypeStruct((n_seed, DIM), F32),
        mesh=mesh,
        scratch_types=[
            pltpu.VMEM((CH,), I32),
            pltpu.VMEM((CH, DIM), F32),
            pltpu.VMEM((CH, DIM), F32),
            pltpu.VMEM((CH, DIM), F32),
            pltpu.VMEM((DIM,), F32),
            pltpu.SemaphoreType.DMA,
            pltpu.SemaphoreType.DMA,
            pltpu.SemaphoreType.DMA,
        ],
    )
    def k2(seed_ref, pa_ref, pb_ref, root_ref, bias_ref, h_ref,
           sidx, acc, tmp, tmp2, bvec, sem0, sem1, sem2):
        c = lax.axis_index("c")
        s = lax.axis_index("s")
        wid = s * NC + c
        pltpu.sync_copy(bias_ref, bvec)
        for ch in range(per_w // CH):
            base = wid * per_w + ch * CH
            pltpu.sync_copy(seed_ref.at[pl.ds(base, CH)], sidx)
            pltpu.async_copy(pa_ref.at[sidx], acc, sem0).wait()
            pltpu.async_copy(pb_ref.at[sidx], tmp, sem1).wait()
            pltpu.async_copy(root_ref.at[sidx], tmp2, sem2).wait()

            def _add(e, carry):
                for k in range(DIM // 16):
                    col = pl.ds(k * 16, 16)
                    acc[e, col] = (acc[e, col] + tmp[e, col]
                                   + tmp2[e, col] + bvec[col])
                return carry

            lax.fori_loop(0, CH, _add, 0, unroll=4)
            pltpu.sync_copy(acc, h_ref.at[pl.ds(base, CH)])

    return k2(seeds, pa, pb, root, bias)


def _head(pa, pb, root, bias2d, h, attn_a, attn_b, labels2d, out_bias2d):
    def body(p0_ref, p1_ref, root_ref, bias_ref, h_ref, a_ref, b_ref,
             lab_ref, ob_ref, scores_ref, loss_ref):
        mf = (p0_ref[...][:N_MOVIE] + p1_ref[...][:N_MOVIE]
              + root_ref[...][:N_MOVIE] + bias_ref[...])
        hm = h_ref[...]
        t = jnp.tanh(jnp.dot(hm, a_ref[...], preferred_element_type=F32))
        e = jnp.dot(t, b_ref[...], preferred_element_type=F32)
        r_ids = lax.broadcasted_iota(I32, (B, B * S), 0)
        c_ids = lax.broadcasted_iota(I32, (B, B * S), 1)
        g = jnp.where(c_ids // S == r_ids, 1.0, 0.0).astype(F32)
        w = jnp.exp(e)
        denom = jnp.dot(g, w, preferred_element_type=F32)
        u = jnp.dot(g, w * hm, preferred_element_type=F32) / denom
        scores = lax.dot_general(u, mf, (((1,), (1,)), ((), ())),
                                 preferred_element_type=F32) + ob_ref[...]
        scores_ref[...] = scores
        m = jnp.max(scores, axis=1, keepdims=True)
        lse = jnp.log(jnp.sum(jnp.exp(scores - m), axis=1, keepdims=True)) + m
        logp = scores - lse
        col = lax.broadcasted_iota(I32, (B, N_MOVIE), 1)
        picked = jnp.sum(jnp.where(col == lab_ref[...], logp, 0.0),
                         axis=1, keepdims=True)
        loss = -jnp.sum(picked) / B
        loss_ref[...] = jnp.broadcast_to(loss, (1, 1))

    return pl.pallas_call(
        body,
        out_shape=(
            jax.ShapeDtypeStruct((B, N_MOVIE), F32),
            jax.ShapeDtypeStruct((1, 1), F32),
        ),
    )(pa, pb, root, bias2d, h, attn_a, attn_b, labels2d, out_bias2d)


def kernel(seed_sets, labels, edge_index, edge_type, weight, root,
           rgcn_bias, attn_a, attn_b, out_bias):
    src = edge_index[0]
    dst = edge_index[1]
    wflat = weight.reshape(N_REL * N_ENTITY, DIM)
    pa, pb = _rgcn_partials(src, dst, edge_type, wflat)
    h = _seed_gather(seed_sets.reshape(-1), pa, pb, root, rgcn_bias)
    scores, loss = _head(pa, pb, root, rgcn_bias.reshape(1, DIM), h,
                         attn_a, attn_b, labels.reshape(B, 1),
                         out_bias.reshape(1, N_MOVIE))
    base_loss = loss.reshape(())
    return scores, base_loss, base_loss


# speedup vs baseline: 1.0482x; 1.0076x over previous
import functools

import jax
import jax.numpy as jnp
from jax import lax
from jax.experimental import pallas as pl
from jax.experimental.pallas import tpu as pltpu
from jax.experimental.pallas import tpu_sc as plsc

N_ENTITY = 10000
N_REL = 12
DIM = 128
N_MOVIE = 6924
N_EDGE = 320000
B = 256
S = 32

NC = 2
NS = 16
CH = 128
SEG = N_ENTITY * N_REL
SEG_PAD = 16 * 7552
CHUNKS_ALL = N_EDGE // CH
CHUNKS_SC = CHUNKS_ALL // NC
F32 = jnp.float32
I32 = jnp.int32


def _rgcn_partials(src, dst, rel, wflat):
    mesh = plsc.VectorSubcoreMesh(core_axis_name="c", subcore_axis_name="s")
    CB = 1024
    NB = 320
    SUP = CHUNKS_SC
    NQ = 80
    EPC = CHUNKS_SC * CH

    @functools.partial(
        pl.kernel,
        out_type=(
            jax.ShapeDtypeStruct((N_ENTITY, DIM), F32),
            jax.ShapeDtypeStruct((N_ENTITY, DIM), F32),
        ),
        mesh=mesh,
        scratch_types=[
            pltpu.VMEM_SHARED((SEG_PAD,), F32),
            pltpu.VMEM_SHARED((N_ENTITY, DIM), F32),
            pltpu.VMEM((CB,), I32),
            pltpu.VMEM((CB,), I32),
            pltpu.VMEM((8, CH), I32),
            pltpu.VMEM((2, CH), I32),
            pltpu.VMEM((2, CH), I32),
            pltpu.VMEM((2, CH), I32),
            pltpu.VMEM((2, CH), I32),
            pltpu.VMEM((2, CH), I32),
            pltpu.VMEM((2, CH), I32),
            pltpu.VMEM((2, CH), F32),
            pltpu.VMEM((144,), F32),
            pltpu.VMEM((144,), F32),
            pltpu.VMEM((CH,), F32),
            pltpu.VMEM((1888,), F32),
            pltpu.VMEM((2 * CH, DIM), F32),
            pltpu.SemaphoreType.DMA,
            pltpu.SemaphoreType.DMA,
            pltpu.SemaphoreType.DMA,
            pltpu.SemaphoreType.DMA,
            pltpu.SemaphoreType.DMA,
            pltpu.SemaphoreType.DMA,
            pltpu.SemaphoreType.DMA,
        ],
    )
    def k1(src_ref, dst_ref, rel_ref, wflat_ref, pa_ref, pb_ref,
           cnt_sh, agg_sh, cdst, crel, cseg, esrc, edst, erel,
           gidx, segidx, dstidx, cval, scale0, scale1, ones, cstage, rows,
           sem_e0, sem_e1, sem_g0, sem_g1, sem_s0, sem_s1, sem_c):
        c = lax.axis_index("c")
        s = lax.axis_index("s")
        sem_e = (sem_e0, sem_e1)
        sem_g = (sem_g0, sem_g1)
        sem_s = (sem_s0, sem_s1)

        for k in range(CH // 16):
            ones[pl.ds(k * 16, 16)] = jnp.ones((16,), F32)

        def _zrow(i, carry):
            for k in range(DIM // 16):
                rows[i, pl.ds(k * 16, 16)] = jnp.zeros((16,), F32)
            return carry

        lax.fori_loop(0, 80, _zrow, 0)

        def _zc(i, carry):
            cstage[pl.ds(i * 16, 16)] = jnp.zeros((16,), F32)
            return carry

        lax.fori_loop(0, 1888 // 16, _zc, 0)

        for j in range(8):
            cid = s + NS * j

            @pl.when(cid < N_ENTITY // 80)
            def _():
                pltpu.sync_copy(rows.at[pl.ds(0, 80)],
                                agg_sh.at[pl.ds(cid * 80, 80)])
        for j in range(4):
            pltpu.sync_copy(cstage, cnt_sh.at[pl.ds(s * 7552 + j * 1888,
                                                    1888)])
        plsc.subcore_barrier()

        def _cblk(jb, carry):
            b = s + NS * jb
            orig = b * CB
            bbase = jnp.minimum(orig, N_EDGE - CB)
            e1 = pltpu.async_copy(dst_ref.at[pl.ds(bbase, CB)], cdst,
                                  sem_e[0])
            e2 = pltpu.async_copy(rel_ref.at[pl.ds(bbase, CB)], crel,
                                  sem_e[1])
            e1.wait()
            e2.wait()
            prev = None
            for k in range(CB // CH):
                for g in range(8):
                    off = k * CH + g * 16
                    d16 = cdst[pl.ds(off, 16)]
                    r16 = crel[pl.ds(off, 16)]
                    seg = d16 * N_REL + r16
                    ci = ((bbase + off) >= orig).astype(I32)
                    cseg[k, pl.ds(g * 16, 16)] = seg * ci + (1 - ci) * SEG
                if prev is not None:
                    prev.wait()
                prev = pltpu.async_copy(ones, cnt_sh.at[cseg.at[k]], sem_c,
                                        add=True)
            prev.wait()
            return carry

        lax.fori_loop(0, NB // NS, _cblk, 0)
        plsc.subcore_barrier()

        for j in range(4):
            pltpu.sync_copy(cnt_sh.at[pl.ds(s * 7552 + j * 1888, 1888)],
                            cstage)

            def _inv(i, carry):
                v = cstage[pl.ds(i * 16, 16)]
                cstage[pl.ds(i * 16, 16)] = 1.0 / jnp.maximum(v, 1.0)
                return carry

            lax.fori_loop(0, 1888 // 16, _inv, 0, unroll=4)
            pltpu.sync_copy(cstage, cnt_sh.at[pl.ds(s * 7552 + j * 1888,
                                                    1888)])
        plsc.subcore_barrier()

        def _pair(q2, carry):
            eds = []
            for sl in (0, 1):
                su = s + NS * (2 * q2 + sl)
                base = c * EPC + jnp.minimum(su, SUP - 1) * CH
                eds.append((
                    pltpu.async_copy(src_ref.at[pl.ds(base, CH)],
                                     esrc.at[sl], sem_e[sl]),
                    pltpu.async_copy(dst_ref.at[pl.ds(base, CH)],
                                     edst.at[sl], sem_e[sl]),
                    pltpu.async_copy(rel_ref.at[pl.ds(base, CH)],
                                     erel.at[sl], sem_e[sl]),
                ))
            gds = []
            for sl in (0, 1):
                for d in eds[sl]:
                    d.wait()
                for g in range(8):
                    off = g * 16
                    s16 = esrc[sl, pl.ds(off, 16)]
                    d16 = edst[sl, pl.ds(off, 16)]
                    r16 = erel[sl, pl.ds(off, 16)]
                    gidx[sl, pl.ds(off, 16)] = r16 * N_ENTITY + s16
                    segidx[sl, pl.ds(off, 16)] = d16 * N_REL + r16
                    dstidx[sl, pl.ds(off, 16)] = d16
                gds.append((
                    pltpu.async_copy(cnt_sh.at[segidx.at[sl]], cval.at[sl],
                                     sem_c),
                    pltpu.async_copy(wflat_ref.at[gidx.at[sl]],
                                     rows.at[pl.ds(sl * CH, CH)],
                                     sem_g[sl]),
                ))
            prev_scatter = None
            for sl in (0, 1):
                gds[sl][0].wait()
                gds[sl][1].wait()
                su = s + NS * (2 * q2 + sl)
                vf = (su < SUP).astype(F32)
                scale = scale0 if sl == 0 else scale1
                for g in range(8):
                    v16 = cval[sl, pl.ds(g * 16, 16)]
                    scale[pl.ds(g * 16, 16)] = v16 * vf

                def _scl(e, carry2):
                    sv = scale[pl.ds(e, 16)][0]
                    for g in range(8):
                        rows[sl * CH + e, pl.ds(g * 16, 16)] = (
                            rows[sl * CH + e, pl.ds(g * 16, 16)] * sv)
                    return carry2

                lax.fori_loop(0, CH, _scl, 0, unroll=8)
                if prev_scatter is not None:
                    prev_scatter.wait()
                prev_scatter = pltpu.async_copy(
                    rows.at[pl.ds(sl * CH, CH)], agg_sh.at[dstidx.at[sl]],
                    sem_s[sl], add=True)
            prev_scatter.wait()
            return carry

        lax.fori_loop(0, NQ // 2, _pair, 0)
        plsc.subcore_barrier()

        for j in range(8):
            cid = s + NS * j

            @pl.when(cid < N_ENTITY // 80)
            def _():
                @pl.when(c == 0)
                def _():
                    pltpu.sync_copy(agg_sh.at[pl.ds(cid * 80, 80)],
                                    pa_ref.at[pl.ds(cid * 80, 80)])

                @pl.when(c == 1)
                def _():
                    pltpu.sync_copy(agg_sh.at[pl.ds(cid * 80, 80)],
                                    pb_ref.at[pl.ds(cid * 80, 80)])

    return k1(src, dst, rel, wflat)


def _seed_gather(seeds, pa, pb, root, bias):
    mesh = plsc.VectorSubcoreMesh(core_axis_name="c", subcore_axis_name="s")
    n_seed = B * S
    per_w = n_seed // (NC * NS)

    @functools.partial(
        pl.kernel,
        out_type=jax.ShapeDtypeStruct((n_seed, DIM), F32),
        mesh=mesh,
        scratch_types=[
            pltpu.VMEM((CH,), I32),
            pltpu.VMEM((CH, DIM), F32),
            pltpu.VMEM((CH, DIM), F32),
            pltpu.VMEM((CH, DIM), F32),
            pltpu.VMEM((DIM,), F32),
            pltpu.SemaphoreType.DMA,
            pltpu.SemaphoreType.DMA,
            pltpu.SemaphoreType.DMA,
        ],
    )
    def k2(seed_ref, pa_ref, pb_ref, root_ref, bias_ref, h_ref,
           sidx, acc, tmp, tmp2, bvec, sem0, sem1, sem2):
        c = lax.axis_index("c")
        s = lax.axis_index("s")
        wid = s * NC + c
        pltpu.sync_copy(bias_ref, bvec)
        for ch in range(per_w // CH):
            base = wid * per_w + ch * CH
            pltpu.sync_copy(seed_ref.at[pl.ds(base, CH)], sidx)
            pltpu.async_copy(pa_ref.at[sidx], acc, sem0).wait()
            pltpu.async_copy(pb_ref.at[sidx], tmp, sem1).wait()
            pltpu.async_copy(root_ref.at[sidx], tmp2, sem2).wait()

            def _add(e, carry):
                for k in range(DIM // 16):
                    col = pl.ds(k * 16, 16)
                    acc[e, col] = (acc[e, col] + tmp[e, col]
                                   + tmp2[e, col] + bvec[col])
                return carry

            lax.fori_loop(0, CH, _add, 0, unroll=4)
            pltpu.sync_copy(acc, h_ref.at[pl.ds(base, CH)])

    return k2(seeds, pa, pb, root, bias)


def _head(pa, pb, root, bias2d, h, attn_a, attn_b, labels2d, out_bias2d):
    def body(p0_ref, p1_ref, root_ref, bias_ref, h_ref, a_ref, b_ref,
             lab_ref, ob_ref, scores_ref, loss_ref):
        mf = (p0_ref[...][:N_MOVIE] + p1_ref[...][:N_MOVIE]
              + root_ref[...][:N_MOVIE] + bias_ref[...])
        hm = h_ref[...]
        t = jnp.tanh(jnp.dot(hm, a_ref[...], preferred_element_type=F32))
        e = jnp.dot(t, b_ref[...], preferred_element_type=F32)
        r_ids = lax.broadcasted_iota(I32, (B, B * S), 0)
        c_ids = lax.broadcasted_iota(I32, (B, B * S), 1)
        g = jnp.where(c_ids // S == r_ids, 1.0, 0.0).astype(F32)
        w = jnp.exp(e)
        denom = jnp.dot(g, w, preferred_element_type=F32)
        u = jnp.dot(g, w * hm, preferred_element_type=F32) / denom
        scores = lax.dot_general(u, mf, (((1,), (1,)), ((), ())),
                                 preferred_element_type=F32) + ob_ref[...]
        scores_ref[...] = scores
        m = jnp.max(scores, axis=1, keepdims=True)
        lse = jnp.log(jnp.sum(jnp.exp(scores - m), axis=1, keepdims=True)) + m
        logp = scores - lse
        col = lax.broadcasted_iota(I32, (B, N_MOVIE), 1)
        picked = jnp.sum(jnp.where(col == lab_ref[...], logp, 0.0),
                         axis=1, keepdims=True)
        loss = -jnp.sum(picked) / B
        loss_ref[...] = jnp.broadcast_to(loss, (1, 1))

    return pl.pallas_call(
        body,
        out_shape=(
            jax.ShapeDtypeStruct((B, N_MOVIE), F32),
            jax.ShapeDtypeStruct((1, 1), F32),
        ),
    )(pa, pb, root, bias2d, h, attn_a, attn_b, labels2d, out_bias2d)


def kernel(seed_sets, labels, edge_index, edge_type, weight, root,
           rgcn_bias, attn_a, attn_b, out_bias):
    src = edge_index[0]
    dst = edge_index[1]
    wflat = weight.reshape(N_REL * N_ENTITY, DIM)
    pa, pb = _rgcn_partials(src, dst, edge_type, wflat)
    h = _seed_gather(seed_sets.reshape(-1), pa, pb, root, rgcn_bias)
    scores, loss = _head(pa, pb, root, rgcn_bias.reshape(1, DIM), h,
                         attn_a, attn_b, labels.reshape(B, 1),
                         out_bias.reshape(1, N_MOVIE))
    base_loss = loss.reshape(())
    return scores, base_loss, base_loss
